# Initial kernel scaffold; baseline (speedup 1.0000x reference)
import jax
import jax.numpy as jnp
from jax.experimental import pallas as pl


def kernel(pred_logits, pred_points, tgt_labels, tgt_points):
    raise NotImplementedError("write your pallas kernel here")


# speedup vs baseline: 2.0792x; 2.0792x over previous
import functools

import jax
import jax.numpy as jnp
from jax import lax
from jax.experimental import pallas as pl
from jax.experimental.pallas import tpu as pltpu

_NUM_CLASSES = 128
_ALPHA = 0.25
_GAMMA = 2.0
_W_CE = 2.0
_W_POINT = 5.0


def _loss_body(logits_ref, labels_ref, pp_ref, tp_ref, out_ref, *, G, B):
    b = pl.program_id(0)

    @pl.when(b == 0)
    def _init():
        out_ref[0] = 0.0
        out_ref[1] = 0.0

    l = logits_ref[0]
    ab = jnp.abs(l)
    e = jnp.exp(-ab)
    lg = jnp.log1p(e)
    inv = 1.0 / (1.0 + e)
    p = jnp.where(l >= 0.0, inv, e * inv)
    relu = jnp.maximum(l, 0.0)
    neg = (1.0 - _ALPHA) * (p * p) * (relu + lg)
    focal_sum = jnp.sum(neg)

    lm = l[:G]
    pm = p[:G]
    ce_pos = relu[:G] - lm + lg[:G]
    pos = _ALPHA * ((1.0 - pm) * (1.0 - pm)) * ce_pos
    lane = lax.broadcasted_iota(jnp.int32, (G, _NUM_CLASSES), 1)
    t = labels_ref[0] == lane
    corr = jnp.sum(jnp.where(t, pos - neg[:G], 0.0))

    d = pp_ref[0][:, : 2 * G] - tp_ref[0]
    ad = jnp.abs(d)
    sl1 = jnp.where(ad < 1.0, 0.5 * d * d, ad - 0.5)
    point_sum = jnp.sum(sl1)

    out_ref[0] += focal_sum + corr
    out_ref[1] += point_sum

    @pl.when(b == B - 1)
    def _finalize():
        np_ = float(B * G)
        out_ref[0] = out_ref[0] * (_W_CE / np_)
        out_ref[1] = out_ref[1] * (_W_POINT / np_)


def kernel(pred_logits, pred_points, tgt_labels, tgt_points):
    B, Q, C = pred_logits.shape
    G = tgt_labels.shape[1]
    labels = tgt_labels.astype(jnp.int32).reshape(B, G, 1)
    pp = pred_points.reshape(B, 1, Q * 2)
    tp = tgt_points.reshape(B, 1, G * 2)

    out = pl.pallas_call(
        functools.partial(_loss_body, G=G, B=B),
        grid=(B,),
        in_specs=[
            pl.BlockSpec((1, Q, C), lambda b: (b, 0, 0)),
            pl.BlockSpec((1, G, 1), lambda b: (b, 0, 0)),
            pl.BlockSpec((1, 1, Q * 2), lambda b: (b, 0, 0)),
            pl.BlockSpec((1, 1, G * 2), lambda b: (b, 0, 0)),
        ],
        out_specs=pl.BlockSpec(memory_space=pltpu.SMEM),
        out_shape=jax.ShapeDtypeStruct((2,), jnp.float32),
    )(pred_logits, labels, pp, tp)
    return out
